# baseline (speedup 1.0000x reference)
import functools

import jax
import jax.numpy as jnp
from jax import lax
from jax.experimental import pallas as pl
from jax.experimental.pallas import tpu as pltpu
from jax.experimental.pallas import tpu_sc as plsc

EMBED_DIM = 32

_NUM_CORES = 2
_NUM_SUBCORES = 16
_NUM_WORKERS = _NUM_CORES * _NUM_SUBCORES

_CHUNK = 1600


def _gather_kernel(idx_hbm, table_hbm, out_hbm, idx_v, rows_v,
                   isem0, isem1, gsem0, gsem1, wsem0, wsem1, *,
                   b_per_w, n_chunks):
    wid = lax.axis_index("s") * _NUM_CORES + lax.axis_index("c")
    base = wid * b_per_w
    isems = (isem0, isem1)
    gsems = (gsem0, gsem1)
    wsems = (wsem0, wsem1)

    def idx_start(g):
        b = g & 1
        return pltpu.async_copy(
            idx_hbm.at[pl.ds(base + g * _CHUNK, _CHUNK)], idx_v.at[b],
            isems[b])

    def gather_start(g):
        b = g & 1
        return pltpu.async_copy(
            table_hbm.at[idx_v.at[b]], rows_v.at[b], gsems[b])

    def wb_start(g):
        b = g & 1
        return pltpu.async_copy(
            rows_v.at[b], out_hbm.at[pl.ds(base + g * _CHUNK, _CHUNK)],
            wsems[b])

    cp_idx = [None] * n_chunks
    cp_g = [None] * n_chunks
    cp_w = [None] * n_chunks

    cp_idx[0] = idx_start(0)
    if n_chunks > 1:
        cp_idx[1] = idx_start(1)
    cp_idx[0].wait()
    cp_g[0] = gather_start(0)

    for g in range(n_chunks):
        cp_g[g].wait()
        if g + 2 < n_chunks:
            cp_idx[g + 2] = idx_start(g + 2)
        cp_w[g] = wb_start(g)
        if g + 1 < n_chunks:
            if g >= 1:
                cp_w[g - 1].wait()
            cp_idx[g + 1].wait()
            cp_g[g + 1] = gather_start(g + 1)

    if n_chunks > 1:
        cp_w[n_chunks - 2].wait()
    cp_w[n_chunks - 1].wait()


def kernel(indices, embed_weight):
    batch, hist = indices.shape
    total = batch * hist
    assert total % (_NUM_WORKERS * _CHUNK) == 0
    b_per_w = total // _NUM_WORKERS
    n_chunks = b_per_w // _CHUNK

    idx_flat = indices.reshape(total).astype(jnp.int32)

    mesh = plsc.VectorSubcoreMesh(core_axis_name="c", subcore_axis_name="s")
    run = functools.partial(
        pl.kernel,
        mesh=mesh,
        compiler_params=pltpu.CompilerParams(use_tc_tiling_on_sc=False),
        out_type=jax.ShapeDtypeStruct((total, EMBED_DIM), jnp.float32),
        scratch_types=[
            pltpu.VMEM((2, _CHUNK), jnp.int32),
            pltpu.VMEM((2, _CHUNK, EMBED_DIM), jnp.float32),
            pltpu.SemaphoreType.DMA,
            pltpu.SemaphoreType.DMA,
            pltpu.SemaphoreType.DMA,
            pltpu.SemaphoreType.DMA,
            pltpu.SemaphoreType.DMA,
            pltpu.SemaphoreType.DMA,
        ],
    )(functools.partial(_gather_kernel, b_per_w=b_per_w, n_chunks=n_chunks))

    out = run(idx_flat, embed_weight)
    return out.reshape(batch, hist, EMBED_DIM)
